# Initial kernel scaffold; baseline (speedup 1.0000x reference)
import jax
import jax.numpy as jnp
from jax.experimental import pallas as pl


def kernel(hidden_states, cu_seqlens, Wq, Wk, temperature, boundary_bias):
    raise NotImplementedError("write your pallas kernel here")


# speedup vs baseline: 1.3183x; 1.3183x over previous
import functools

import jax
import jax.numpy as jnp
from jax import lax
from jax.experimental import pallas as pl
from jax.experimental.pallas import tpu as pltpu

BLK = 512


def _routing_block(cs_ref, scal_ref, hs_ref, prev8_ref, wq_ref, wk_ref,
                   bp_ref, mask_ref, sp_ref):
    b = pl.program_id(0)
    cur = hs_ref[...]
    prev_row = prev8_ref[7:8, :]
    rolled = pltpu.roll(cur, shift=1, axis=0)
    row_iota = lax.broadcasted_iota(jnp.int32, (BLK, 1), 0)
    shifted = jnp.where(row_iota == 0, prev_row, rolled)

    dims = (((1,), (1,)), ((), ()))
    q = lax.dot_general(shifted, wq_ref[...], dims,
                        preferred_element_type=jnp.float32)
    k = lax.dot_general(cur, wk_ref[...], dims,
                        preferred_element_type=jnp.float32)

    dot = jnp.sum(q * k, axis=1, keepdims=True)
    qn = jnp.maximum(jnp.sqrt(jnp.sum(q * q, axis=1, keepdims=True)), 1e-12)
    kn = jnp.maximum(jnp.sqrt(jnp.sum(k * k, axis=1, keepdims=True)), 1e-12)
    cos = dot / (qn * kn)

    temp = jnp.clip(jnp.abs(scal_ref[0]), 0.1, 2.0)
    logits = (1.0 - cos + scal_ref[1]) / temp
    p = jax.nn.sigmoid(logits)

    gid = row_iota + b * BLK
    force = gid == 0
    for j in range(16):
        force = jnp.logical_or(force, gid == cs_ref[j])
    p = jnp.where(force, 1.0, p)

    one_m = 1.0 - p
    bp_ref[...] = jnp.concatenate([one_m, p], axis=1)
    m = p > 0.5
    mask_ref[...] = m.astype(jnp.int8)
    sp_ref[...] = jnp.where(m, p, one_m)


@functools.partial(jax.jit, static_argnames=())
def kernel(hidden_states, cu_seqlens, Wq, Wk, temperature, boundary_bias):
    T, D = hidden_states.shape
    grid = (T // BLK,)
    scal = jnp.stack([temperature.astype(jnp.float32),
                      boundary_bias.astype(jnp.float32)])
    bp, mask8, sp = pl.pallas_call(
        _routing_block,
        grid=grid,
        in_specs=[
            pl.BlockSpec(memory_space=pltpu.SMEM),
            pl.BlockSpec(memory_space=pltpu.SMEM),
            pl.BlockSpec((BLK, D), lambda i: (i, 0)),
            pl.BlockSpec((8, D),
                         lambda i: (lax.max(i * (BLK // 8) - 1, 0), 0)),
            pl.BlockSpec((D, D), lambda i: (0, 0)),
            pl.BlockSpec((D, D), lambda i: (0, 0)),
        ],
        out_specs=[
            pl.BlockSpec((BLK, 2), lambda i: (i, 0)),
            pl.BlockSpec((BLK, 1), lambda i: (i, 0)),
            pl.BlockSpec((BLK, 1), lambda i: (i, 0)),
        ],
        out_shape=[
            jax.ShapeDtypeStruct((T, 2), jnp.float32),
            jax.ShapeDtypeStruct((T, 1), jnp.int8),
            jax.ShapeDtypeStruct((T, 1), jnp.float32),
        ],
        compiler_params=pltpu.CompilerParams(
            dimension_semantics=("arbitrary",),
        ),
    )(cu_seqlens, scal, hidden_states, hidden_states, Wq, Wk)
    return bp, mask8.reshape(T).astype(jnp.bool_), sp
